# baseline (speedup 1.0000x reference)
import jax
import jax.numpy as jnp
from jax.experimental import pallas as pl
from jax.experimental.pallas import tpu as pltpu

_BATCH = 4096
_D_IN = 1024
_D_LEARNT = 8192
_N_DEAD = 512

_BN = 256


def _scatter_body(idx_ref, upd_ref, w_in_ref, w_out_ref):
    del idx_ref, w_in_ref
    w_out_ref[...] = upd_ref[...]


def _matmul_body(x_ref, w_ref, b_ref, o_ref):
    acc = jax.lax.dot_general(
        x_ref[...], w_ref[...],
        dimension_numbers=(((1,), (1,)), ((), ())),
        preferred_element_type=jnp.float32,
    )
    o_ref[...] = jnp.maximum(acc + b_ref[...], 0.0)


def kernel(x, dictionary_vector_indices, updated_dictionary_weights, weight, bias):
    idx = dictionary_vector_indices.astype(jnp.int32)

    order = jnp.arange(_N_DEAD, dtype=jnp.int32)
    eq = idx[:, None] == idx[None, :]
    winner = jnp.max(jnp.where(eq, order[None, :], -1), axis=1)
    upd = updated_dictionary_weights[winner]

    upd3 = upd.reshape(_N_DEAD, 1, _D_IN)
    w3 = weight.reshape(_D_LEARNT, 1, _D_IN)
    w_new = pl.pallas_call(
        _scatter_body,
        grid_spec=pltpu.PrefetchScalarGridSpec(
            num_scalar_prefetch=1,
            grid=(_N_DEAD,),
            in_specs=[
                pl.BlockSpec((1, 1, _D_IN), lambda i, idx_ref: (i, 0, 0)),
                pl.BlockSpec((1, 1, _D_IN), lambda i, idx_ref: (idx_ref[i], 0, 0)),
            ],
            out_specs=pl.BlockSpec((1, 1, _D_IN), lambda i, idx_ref: (idx_ref[i], 0, 0)),
        ),
        out_shape=jax.ShapeDtypeStruct((_D_LEARNT, 1, _D_IN), jnp.float32),
        input_output_aliases={2: 0},
    )(idx, upd3, w3).reshape(_D_LEARNT, _D_IN)

    bias2 = bias.reshape(1, _D_LEARNT)
    out = pl.pallas_call(
        _matmul_body,
        grid=(_D_LEARNT // _BN,),
        in_specs=[
            pl.BlockSpec((_BATCH, _D_IN), lambda j: (0, 0)),
            pl.BlockSpec((_BN, _D_IN), lambda j: (j, 0)),
            pl.BlockSpec((1, _BN), lambda j: (0, j)),
        ],
        out_specs=pl.BlockSpec((_BATCH, _BN), lambda j: (0, j)),
        out_shape=jax.ShapeDtypeStruct((_BATCH, _D_LEARNT), jnp.float32),
    )(x, w_new, bias2)
    return out


# speedup vs baseline: 3.2834x; 3.2834x over previous
import jax
import jax.numpy as jnp
from jax.experimental import pallas as pl
from jax.experimental.pallas import tpu as pltpu

_BATCH = 4096
_D_IN = 1024
_D_LEARNT = 8192
_N_DEAD = 512

_BN = 256


def _fused_body(idx_ref, starts_ref, x_ref, w_ref, upd_ref, b_ref, o_ref, w_scr):
    j = pl.program_id(0)
    w_scr[...] = w_ref[...]
    s0 = starts_ref[j]
    s1 = starts_ref[j + 1]

    def patch(s, carry):
        r = idx_ref[s] - j * _BN
        w_scr[pl.ds(r, 1), :] = upd_ref[pl.ds(s, 1), :]
        return carry

    jax.lax.fori_loop(s0, s1, patch, 0)

    acc = jax.lax.dot_general(
        x_ref[...], w_scr[...],
        dimension_numbers=(((1,), (1,)), ((), ())),
        preferred_element_type=jnp.float32,
    )
    o_ref[...] = jnp.maximum(acc + b_ref[...], 0.0)


def kernel(x, dictionary_vector_indices, updated_dictionary_weights, weight, bias):
    idx = dictionary_vector_indices.astype(jnp.int32)

    order = jnp.arange(_N_DEAD, dtype=jnp.int32)
    eq = idx[:, None] == idx[None, :]
    winner = jnp.max(jnp.where(eq, order[None, :], -1), axis=1)
    upd = updated_dictionary_weights[winner]

    perm = jnp.argsort(idx)
    idx_s = idx[perm]
    upd_s = upd[perm]
    starts = jnp.searchsorted(
        idx_s, jnp.arange(_D_LEARNT // _BN + 1, dtype=jnp.int32) * _BN
    ).astype(jnp.int32)

    bias2 = bias.reshape(1, _D_LEARNT)
    out = pl.pallas_call(
        _fused_body,
        grid_spec=pltpu.PrefetchScalarGridSpec(
            num_scalar_prefetch=2,
            grid=(_D_LEARNT // _BN,),
            in_specs=[
                pl.BlockSpec((_BATCH, _D_IN), lambda j, i_r, s_r: (0, 0)),
                pl.BlockSpec((_BN, _D_IN), lambda j, i_r, s_r: (j, 0)),
                pl.BlockSpec((_N_DEAD, _D_IN), lambda j, i_r, s_r: (0, 0)),
                pl.BlockSpec((1, _BN), lambda j, i_r, s_r: (0, j)),
            ],
            out_specs=pl.BlockSpec((_BATCH, _BN), lambda j, i_r, s_r: (0, j)),
            scratch_shapes=[pltpu.VMEM((_BN, _D_IN), jnp.float32)],
        ),
        out_shape=jax.ShapeDtypeStruct((_BATCH, _D_LEARNT), jnp.float32),
    )(idx_s, starts, x, weight, upd_s, bias2)
    return out
